# baseline (speedup 1.0000x reference)
import functools

import jax
import jax.numpy as jnp
from jax import lax
from jax.experimental import pallas as pl
from jax.experimental.pallas import tpu as pltpu
from jax.experimental.pallas import tpu_sc as plsc

B, S = 4096, 50
SP = 56
D = 256
O = 64
R2 = B // 2
C2 = 2 * SP
NC, NS = 2, 16
NW = NC * NS
ROWS_PER_W = R2 // NW


@functools.partial(
    pl.kernel,
    out_type=jax.ShapeDtypeStruct((R2, C2, D), jnp.float32),
    mesh=plsc.VectorSubcoreMesh(core_axis_name="c", subcore_axis_name="s"),
    scratch_types=[
        pltpu.VMEM((ROWS_PER_W, C2), jnp.int32),
        pltpu.VMEM((C2, D), jnp.float32),
        pltpu.SemaphoreType.DMA,
    ],
)
def _sc_gather(table_hbm, x_hbm, h_hbm, idx_v, buf, sem):
    wid = lax.axis_index("s") * NC + lax.axis_index("c")
    r0 = wid * ROWS_PER_W
    pltpu.sync_copy(x_hbm.at[pl.ds(r0, ROWS_PER_W), :], idx_v)

    def body(i, carry):
        pltpu.async_copy(table_hbm.at[idx_v.at[i, :]], buf, sem).wait()
        pltpu.sync_copy(buf, h_hbm.at[r0 + i])
        return carry

    lax.fori_loop(0, ROWS_PER_W, body, 0)


BLK_B = 64
BLK2 = BLK_B // 2


def _tc_matmul_body(h_ref, w_ref, b_ref, o_ref):
    h2 = h_ref[...].reshape(BLK2 * C2, D)
    acc = lax.dot_general(
        h2, w_ref[...], (((1,), (1,)), ((), ())),
        preferred_element_type=jnp.float32,
    ) + b_ref[...]
    o_ref[...] = acc.reshape(BLK_B, SP, O)


def _tc_matmul(h2, W, b):
    return pl.pallas_call(
        _tc_matmul_body,
        grid=(B // BLK_B,),
        in_specs=[
            pl.BlockSpec((BLK2, C2, D), lambda i: (i, 0, 0)),
            pl.BlockSpec((O, D), lambda i: (0, 0)),
            pl.BlockSpec((1, O), lambda i: (0, 0)),
        ],
        out_specs=pl.BlockSpec((BLK_B, SP, O), lambda i: (i, 0, 0)),
        out_shape=jax.ShapeDtypeStruct((B, S, O), jnp.float32),
    )(h2, W, b.reshape(1, O))


def kernel(x, table, W, b):
    xp = jnp.pad(x.astype(jnp.int32), ((0, 0), (0, SP - S)))
    xp2 = xp.reshape(R2, C2)
    h2 = _sc_gather(table, xp2)
    return _tc_matmul(h2, W, b)
